# baseline (speedup 1.0000x reference)
import functools

import jax
import jax.numpy as jnp
from jax import lax
from jax.experimental import pallas as pl
from jax.experimental.pallas import tpu as pltpu
from jax.experimental.pallas import tpu_sc as plsc

NC = 2
NS = 16
NW = NC * NS
L = 16
D = 64


def _rsqrt16(x):
    i = plsc.bitcast(x, jnp.int32)
    z = plsc.bitcast(jnp.int32(0x5F3759DF) - (i >> 1), jnp.float32)
    for _ in range(3):
        z = z * (1.5 - 0.5 * x * z * z)
    return z


def _build_sc_kernel(B):
    CH = B // NW
    NG = CH // L

    mesh = plsc.VectorSubcoreMesh(
        core_axis_name="c", subcore_axis_name="s",
        num_cores=NC, num_subcores=NS)

    @functools.partial(
        pl.kernel,
        out_type=jax.ShapeDtypeStruct((NW, L), jnp.float32),
        mesh=mesh,
        scratch_types=[
            pltpu.VMEM((CH,), jnp.int32),
            pltpu.VMEM((CH,), jnp.int32),
            pltpu.VMEM((CH,), jnp.int32),
            pltpu.VMEM((CH, D), jnp.float32),
            pltpu.VMEM((CH, D), jnp.float32),
            pltpu.VMEM((CH, D), jnp.float32),
            pltpu.VMEM((CH,), jnp.float32),
            pltpu.VMEM((L,), jnp.float32),
            pltpu.SemaphoreType.DMA,
        ],
        compiler_params=pltpu.CompilerParams(
            needs_layout_passes=False, use_tc_tiling_on_sc=False),
    )
    def sc_kernel(ph, pr, pt, nh, nr, nt, node_em, edge_em, out,
                  ih_v, ir_v, it_v, h_v, r_v, t_v, pd_v, acc_v, sem):
        wid = lax.axis_index("s") * NC + lax.axis_index("c")
        base = wid * CH

        def gather_phase(hi, ri, ti):
            pltpu.sync_copy(hi.at[pl.ds(base, CH)], ih_v)
            pltpu.sync_copy(ri.at[pl.ds(base, CH)], ir_v)
            pltpu.sync_copy(ti.at[pl.ds(base, CH)], it_v)
            c1 = pltpu.async_copy(node_em.at[ih_v], h_v, sem)
            c2 = pltpu.async_copy(edge_em.at[ir_v], r_v, sem)
            c3 = pltpu.async_copy(node_em.at[it_v], t_v, sem)
            c1.wait(); c2.wait(); c3.wait()

        def dist_group(g):
            row = g * L + lax.iota(jnp.int32, L)

            def jstep(j, ss):
                col = jnp.full((L,), 0, jnp.int32) + j
                h = plsc.load_gather(h_v, [row, col])
                r = plsc.load_gather(r_v, [row, col])
                t = plsc.load_gather(t_v, [row, col])
                d = h + r - t
                return ss + d * d

            ss = lax.fori_loop(0, D, jstep, jnp.zeros((L,), jnp.float32))
            return ss * _rsqrt16(ss)

        gather_phase(ph, pr, pt)

        def pos_body(g, carry):
            pd_v[pl.ds(g * L, L)] = dist_group(g)
            return carry

        lax.fori_loop(0, NG, pos_body, 0)

        gather_phase(nh, nr, nt)

        def neg_body(g, acc):
            nd = dist_group(g)
            pd = pd_v[pl.ds(g * L, L)]
            return acc + jnp.maximum(pd - nd, 0.0)

        acc = lax.fori_loop(0, NG, neg_body, jnp.zeros((L,), jnp.float32))
        acc_v[...] = acc
        pltpu.sync_copy(acc_v, out.at[wid])

    return sc_kernel


def _finish_body(inv_b, p_ref, o_ref):
    o_ref[...] = jnp.reshape(jnp.sum(p_ref[...]) * inv_b, (1, 1))


def kernel(pos_triplets, neg_triplets, node_em, edge_em):
    B = pos_triplets.shape[0]
    sc = _build_sc_kernel(B)
    partials = sc(
        pos_triplets[:, 0], pos_triplets[:, 1], pos_triplets[:, 2],
        neg_triplets[:, 0], neg_triplets[:, 1], neg_triplets[:, 2],
        node_em, edge_em)
    loss2d = pl.pallas_call(
        functools.partial(_finish_body, 1.0 / B),
        out_shape=jax.ShapeDtypeStruct((1, 1), jnp.float32),
    )(partials)
    return loss2d[0, 0]
